# baseline (speedup 1.0000x reference)
import functools

import jax
import jax.numpy as jnp
from jax import lax
from jax.experimental import pallas as pl
from jax.experimental.pallas import tpu as pltpu
from jax.experimental.pallas import tpu_sc as plsc

NC = 2
NS = 16
LANES = 16
NW = NC * NS
CHUNK = 128
NG = 4
NI = 8


def _embed_kernel(n_fields, vocab, dim, batch):
    pack = 128 // dim
    pack_shift = pack.bit_length() - 1
    win_per_w = (batch // CHUNK) // NW
    n_blocks = (n_fields + pack - 1) // pack
    rem_f = n_fields - (n_blocks - 1) * pack
    mesh = plsc.VectorSubcoreMesh(core_axis_name="c", subcore_axis_name="s")

    @functools.partial(
        pl.kernel,
        mesh=mesh,
        compiler_params=pltpu.CompilerParams(use_tc_tiling_on_sc=True),
        out_type=jax.ShapeDtypeStruct((batch, n_blocks * 128), jnp.float32),
        scratch_types=(
            [pltpu.VMEM((CHUNK,), jnp.int32) for _ in range(NI + NG)]
            + [pltpu.VMEM((CHUNK, 128), jnp.float32) for _ in range(NG)]
            + [pltpu.VMEM((CHUNK, 128), jnp.float32),
               pltpu.VMEM((CHUNK, 128), jnp.float32)]
            + [pltpu.SemaphoreType.DMA for _ in range(NI + NG + 2)]
        ),
    )
    def k(tables_hbm, featsT_hbm, out_hbm, *refs):
        idx_v = refs[:NI]
        idxp_v = refs[NI:NI + NG]
        rows = refs[NI + NG:NI + 2 * NG]
        blks = refs[NI + 2 * NG:NI + 2 * NG + 2]
        sems = refs[NI + 2 * NG + 2:]
        isem = sems[:NI]
        gsem = sems[NI:NI + NG]
        osem = sems[NI + NG:]

        wid = lax.axis_index("s") * NC + lax.axis_index("c")

        def b0_of(w_):
            return pl.multiple_of((wid * win_per_w + w_) * CHUNK, CHUNK)

        def stage_idx(f, w_, s):
            pltpu.make_async_copy(
                featsT_hbm.at[f, pl.ds(b0_of(w_), CHUNK)], idx_v[s], isem[s],
            ).start()

        def fire_gather(f, s, g):
            pltpu.make_async_copy(
                featsT_hbm.at[0, pl.ds(0, CHUNK)], idx_v[s], isem[s],
            ).wait()
            for t in range(CHUNK // LANES):
                sl = pl.ds(t * LANES, LANES)
                idxp_v[g][sl] = jnp.right_shift(idx_v[s][sl], pack_shift)
            pltpu.make_async_copy(
                tables_hbm.at[f].at[idxp_v[g]], rows[g], gsem[g],
            ).start()

        def blk_of(t):
            return blks[t % 2]

        def out_desc(t, w_):
            return pltpu.make_async_copy(
                blk_of(t),
                out_hbm.at[pl.ds(b0_of(w_), CHUNK),
                           pl.ds(pl.multiple_of(t * 128, 128), 128)],
                osem[t % 2],
            )

        def extract(s, g, t, q):
            blk_ref = blk_of(t)

            def body(g16, carry):
                base = g16 * LANES
                off16 = (idx_v[s][pl.ds(base, LANES)] & (pack - 1)) * dim
                for u in range(LANES):
                    r = base + u
                    off = off16[u]
                    for h in range(dim // LANES):
                        blk_ref[r, pl.ds(q * dim + h * LANES, LANES)] = (
                            rows[g][r, pl.ds(off + h * LANES, LANES)])
                return carry

            lax.fori_loop(0, CHUNK // LANES, body, 0)

        def window(w_, carry):
            for c in range(n_fields):
                t, q = c // pack, c % pack
                stage_idx(c, w_, 0)
                fire_gather(c, 0, 0)
                pltpu.make_async_copy(
                    tables_hbm.at[0].at[idxp_v[0]], rows[0], gsem[0],
                ).wait()
                extract(0, 0, t, q)
                if q == pack - 1 or c == n_fields - 1:
                    out_desc(t, w_).start()
                    out_desc(t, w_).wait()
            return carry

        lax.fori_loop(0, win_per_w, window, 0)

    return k


def kernel(features, tables):
    b, f = features.shape
    f2, vocab, dim = tables.shape
    assert f == f2
    pack = 128 // dim
    assert 128 % dim == 0 and vocab % pack == 0 and b % CHUNK == 0
    assert (b // CHUNK) % NW == 0 and dim % LANES == 0 and f > 2 * pack

    feats_t = features.astype(jnp.int32).T
    tables_p = tables.reshape(f, vocab // pack, 128)
    out = _embed_kernel(f, vocab, dim, b)(tables_p, feats_t)
    return out[:, : f * dim]


# speedup vs baseline: 1.1421x; 1.1421x over previous
import functools

import jax
import jax.numpy as jnp
from jax import lax
from jax.experimental import pallas as pl
from jax.experimental.pallas import tpu as pltpu
from jax.experimental.pallas import tpu_sc as plsc

NC = 2
NS = 16
LANES = 16
NW = NC * NS
CHUNK = 128
NG = 4
NI = 8


def _embed_kernel(n_fields, vocab, dim, batch):
    pack = 128 // dim
    pack_shift = pack.bit_length() - 1
    win_per_w = (batch // CHUNK) // NW
    n_blocks = (n_fields + pack - 1) // pack
    rem_f = n_fields - (n_blocks - 1) * pack
    mesh = plsc.VectorSubcoreMesh(core_axis_name="c", subcore_axis_name="s")

    @functools.partial(
        pl.kernel,
        mesh=mesh,
        compiler_params=pltpu.CompilerParams(use_tc_tiling_on_sc=True),
        out_type=jax.ShapeDtypeStruct((batch, n_blocks * 128), jnp.float32),
        scratch_types=(
            [pltpu.VMEM((CHUNK,), jnp.int32) for _ in range(NI + NG)]
            + [pltpu.VMEM((CHUNK, 128), jnp.float32) for _ in range(NG)]
            + [pltpu.VMEM((CHUNK, 128), jnp.float32),
               pltpu.VMEM((CHUNK, 128), jnp.float32)]
            + [pltpu.SemaphoreType.DMA for _ in range(NI + NG + 2)]
        ),
    )
    def k(tables_hbm, featsT_hbm, out_hbm, *refs):
        idx_v = refs[:NI]
        idxp_v = refs[NI:NI + NG]
        rows = refs[NI + NG:NI + 2 * NG]
        blks = refs[NI + 2 * NG:NI + 2 * NG + 2]
        sems = refs[NI + 2 * NG + 2:]
        isem = sems[:NI]
        gsem = sems[NI:NI + NG]
        osem = sems[NI + NG:]

        wid = lax.axis_index("s") * NC + lax.axis_index("c")

        def b0_of(w_):
            return pl.multiple_of((wid * win_per_w + w_) * CHUNK, CHUNK)

        def stage_idx(f, w_, s):
            pltpu.make_async_copy(
                featsT_hbm.at[f, pl.ds(b0_of(w_), CHUNK)], idx_v[s], isem[s],
            ).start()

        def fire_gather(f, s, g):
            pltpu.make_async_copy(
                featsT_hbm.at[0, pl.ds(0, CHUNK)], idx_v[s], isem[s],
            ).wait()
            for t in range(CHUNK // LANES):
                sl = pl.ds(t * LANES, LANES)
                idxp_v[g][sl] = jnp.right_shift(idx_v[s][sl], pack_shift)
            pltpu.make_async_copy(
                tables_hbm.at[f].at[idxp_v[g]], rows[g], gsem[g],
            ).start()

        def blk_of(t):
            return blks[t % 2]

        def out_desc(t, w_):
            return pltpu.make_async_copy(
                blk_of(t),
                out_hbm.at[pl.ds(b0_of(w_), CHUNK),
                           pl.ds(pl.multiple_of(t * 128, 128), 128)],
                osem[t % 2],
            )

        def extract(s, g, t, q):
            blk_ref = blk_of(t)

            def body(g16, carry):
                base = g16 * LANES
                off16 = (idx_v[s][pl.ds(base, LANES)] & (pack - 1)) * dim
                for u in range(LANES):
                    r = base + u
                    off = off16[u]
                    for h in range(dim // LANES):
                        blk_ref[r, pl.ds(q * dim + h * LANES, LANES)] = (
                            rows[g][r, pl.ds(off + h * LANES, LANES)])
                return carry

            lax.fori_loop(0, CHUNK // LANES, body, 0)

        def window(w_, carry):
            for s in range(NI):
                stage_idx(s, w_, s)
            for c in range(NG):
                fire_gather(c, c % NI, c)
            for c in range(n_fields):
                t, q = c // pack, c % pack
                sl, g = c % NI, c % NG
                pltpu.make_async_copy(
                    tables_hbm.at[0].at[idxp_v[g]], rows[g], gsem[g],
                ).wait()

                if q == 0:
                    if t >= 2:
                        out_desc(t - 2, w_).wait()
                    else:
                        tp = n_blocks - 1 if t == (n_blocks - 1) % 2 \
                            else n_blocks - 2
                        @pl.when(w_ > 0)
                        def _():
                            out_desc(tp, w_ - 1).wait()

                extract(sl, g, t, q)

                if q == pack - 1 or c == n_fields - 1:
                    out_desc(t, w_).start()

                ci = c + NI
                if ci < n_fields:
                    stage_idx(ci, w_, ci % NI)
                cg = c + NG
                if cg < n_fields:
                    fire_gather(cg, cg % NI, cg % NG)
            return carry

        lax.fori_loop(0, win_per_w, window, 0)
        for t in (n_blocks - 2, n_blocks - 1):
            out_desc(t, win_per_w - 1).wait()

    return k


def kernel(features, tables):
    b, f = features.shape
    f2, vocab, dim = tables.shape
    assert f == f2
    pack = 128 // dim
    assert 128 % dim == 0 and vocab % pack == 0 and b % CHUNK == 0
    assert (b // CHUNK) % NW == 0 and dim % LANES == 0 and f > 2 * pack

    feats_t = features.astype(jnp.int32).T
    tables_p = tables.reshape(f, vocab // pack, 128)
    out = _embed_kernel(f, vocab, dim, b)(tables_p, feats_t)
    return out[:, : f * dim]
